# baseline (speedup 1.0000x reference)
import functools

import jax
import jax.numpy as jnp
from jax import lax
from jax.experimental import pallas as pl
from jax.experimental.pallas import tpu as pltpu
from jax.experimental.pallas import tpu_sc as plsc

SEQ = 8192
DIM = 4096
NUM_CORES = 2
NUM_SUBCORES = 16
NUM_WORKERS = NUM_CORES * NUM_SUBCORES
ROWS_PER_WORKER = SEQ // NUM_WORKERS

_mesh = plsc.VectorSubcoreMesh(
    core_axis_name="c", subcore_axis_name="s", num_cores=NUM_CORES
)


@functools.partial(
    pl.kernel,
    out_type=jax.ShapeDtypeStruct((SEQ, DIM), jnp.float32),
    mesh=_mesh,
)
def _sc_identity_gather(table_hbm, out_hbm):
    wid = lax.axis_index("s") * NUM_CORES + lax.axis_index("c")
    base = wid * ROWS_PER_WORKER
    pltpu.sync_copy(
        table_hbm.at[pl.ds(base, ROWS_PER_WORKER)],
        out_hbm.at[pl.ds(base, ROWS_PER_WORKER)],
    )


def kernel(x, emb_weight):
    del x
    return _sc_identity_gather(emb_weight)


# speedup vs baseline: 35.8986x; 35.8986x over previous
import functools

import jax
import jax.numpy as jnp
from jax import lax
from jax.experimental import pallas as pl
from jax.experimental.pallas import tpu as pltpu
from jax.experimental.pallas import tpu_sc as plsc

SEQ = 8192
DIM = 4096
NUM_CORES = 2
NUM_SUBCORES = 16
NUM_WORKERS = NUM_CORES * NUM_SUBCORES
ROWS_PER_WORKER = SEQ // NUM_WORKERS

ROWS_PER_CHUNK = 4
NBUF = 4
NCHUNK = ROWS_PER_WORKER // ROWS_PER_CHUNK
NGRP = NCHUNK // NBUF

_mesh = plsc.VectorSubcoreMesh(
    core_axis_name="c", subcore_axis_name="s", num_cores=NUM_CORES
)


@functools.partial(
    pl.kernel,
    out_type=jax.ShapeDtypeStruct((SEQ, DIM), jnp.float32),
    mesh=_mesh,
    scratch_types=[
        pltpu.VMEM((NBUF, ROWS_PER_CHUNK, DIM), jnp.float32),
        pltpu.SemaphoreType.DMA((NBUF,)),
        pltpu.SemaphoreType.DMA((NBUF,)),
    ],
)
def _sc_identity_gather(table_hbm, out_hbm, bufs, load_sems, store_sems):
    wid = lax.axis_index("s") * NUM_CORES + lax.axis_index("c")
    wbase = wid * ROWS_PER_WORKER

    def load_desc(i, b):
        row = wbase + i * ROWS_PER_CHUNK
        return pltpu.make_async_copy(
            table_hbm.at[pl.ds(row, ROWS_PER_CHUNK)], bufs.at[b], load_sems.at[b]
        )

    def store_desc(i, b):
        row = wbase + i * ROWS_PER_CHUNK
        return pltpu.make_async_copy(
            bufs.at[b], out_hbm.at[pl.ds(row, ROWS_PER_CHUNK)], store_sems.at[b]
        )

    for b in range(NBUF):
        load_desc(b, b).start()

    def group(g, carry):
        gbase = g * NBUF
        for b in range(NBUF):
            load_desc(gbase + b, b).wait()
            store_desc(gbase + b, b).start()
        for b in range(NBUF):
            store_desc(gbase + b, b).wait()
            load_desc(gbase + NBUF + b, b).start()
        return carry

    lax.fori_loop(0, NGRP - 1, group, 0)

    gbase = (NGRP - 1) * NBUF
    for b in range(NBUF):
        load_desc(gbase + b, b).wait()
        store_desc(gbase + b, b).start()
    for b in range(NBUF):
        store_desc(gbase + b, b).wait()


def kernel(x, emb_weight):
    del x
    return _sc_identity_gather(emb_weight)


# speedup vs baseline: 36.1794x; 1.0078x over previous
import functools

import jax
import jax.numpy as jnp
from jax import lax
from jax.experimental import pallas as pl
from jax.experimental.pallas import tpu as pltpu
from jax.experimental.pallas import tpu_sc as plsc

SEQ = 8192
DIM = 4096
NUM_CORES = 2
NUM_SUBCORES = 16
NUM_WORKERS = NUM_CORES * NUM_SUBCORES
ROWS_PER_WORKER = SEQ // NUM_WORKERS

ROWS_PER_CHUNK = 2
NBUF = 8
NCHUNK = ROWS_PER_WORKER // ROWS_PER_CHUNK
NGRP = NCHUNK // NBUF

_mesh = plsc.VectorSubcoreMesh(
    core_axis_name="c", subcore_axis_name="s", num_cores=NUM_CORES
)


@functools.partial(
    pl.kernel,
    out_type=jax.ShapeDtypeStruct((SEQ, DIM), jnp.float32),
    mesh=_mesh,
    scratch_types=[
        pltpu.VMEM((NBUF, ROWS_PER_CHUNK, DIM), jnp.float32),
        pltpu.SemaphoreType.DMA((NBUF,)),
        pltpu.SemaphoreType.DMA((NBUF,)),
    ],
)
def _sc_identity_gather(table_hbm, out_hbm, bufs, load_sems, store_sems):
    wid = lax.axis_index("s") * NUM_CORES + lax.axis_index("c")
    wbase = wid * ROWS_PER_WORKER

    def load_desc(i, b):
        row = wbase + i * ROWS_PER_CHUNK
        return pltpu.make_async_copy(
            table_hbm.at[pl.ds(row, ROWS_PER_CHUNK)], bufs.at[b], load_sems.at[b]
        )

    def store_desc(i, b):
        row = wbase + i * ROWS_PER_CHUNK
        return pltpu.make_async_copy(
            bufs.at[b], out_hbm.at[pl.ds(row, ROWS_PER_CHUNK)], store_sems.at[b]
        )

    for b in range(NBUF):
        load_desc(b, b).start()

    def group(g, carry):
        gbase = g * NBUF
        for b in range(NBUF):
            load_desc(gbase + b, b).wait()
            store_desc(gbase + b, b).start()
        for b in range(NBUF):
            store_desc(gbase + b, b).wait()
            load_desc(gbase + NBUF + b, b).start()
        return carry

    lax.fori_loop(0, NGRP - 1, group, 0)

    gbase = (NGRP - 1) * NBUF
    for b in range(NBUF):
        load_desc(gbase + b, b).wait()
        store_desc(gbase + b, b).start()
    for b in range(NBUF):
        store_desc(gbase + b, b).wait()


def kernel(x, emb_weight):
    del x
    return _sc_identity_gather(emb_weight)
